# baseline (speedup 1.0000x reference)
import jax
import jax.numpy as jnp
import numpy as np
from jax import lax
from jax.experimental import pallas as pl
from jax.experimental.pallas import tpu as pltpu
from jax.experimental.pallas import tpu_sc as plsc

NC = 2
NS = 16
NW = NC * NS
LANES = 16

B = 4096
K = 50
KP = 56
CONV = 256
NCH = CONV // LANES
ROWS_PW = B // NW
NBUF = 4
GR = 8
NGRP = ROWS_PW // GR
GW = GR * KP
LOOKAHEAD = 3


def _bag_body(idx_hbm, vals_hbm, table_hbm, y_hbm,
              idx_v, vals_v, bufs, ystage, sems):
    c = lax.axis_index("c")
    s = lax.axis_index("s")
    wid = s * NC + c
    rbase = wid * ROWS_PW

    pltpu.sync_copy(idx_hbm.at[pl.ds(rbase * KP, ROWS_PW * KP)], idx_v)
    pltpu.sync_copy(vals_hbm.at[pl.ds(rbase * KP, ROWS_PW * KP)], vals_v)

    def issue(base, b):
        vs = [idx_v[pl.ds(base + 16 * t, LANES)] for t in range(3)]
        vs.append(idx_v[pl.ds(base + 40, LANES)])
        for j in range(KP):
            t, l = divmod(j, LANES) if j < 48 else (3, j - 40)
            iv = vs[t][l]
            pltpu.async_copy(table_hbm.at[pl.ds(iv, 1), :],
                             bufs[b].at[pl.ds(j, 1), :], sems[b])

    def wait(b):
        pltpu.make_async_copy(table_hbm.at[pl.ds(0, KP), :],
                              bufs[b], sems[b]).wait()

    def accum_row(r, buf):
        def jbody(j, acc):
            w = plsc.load_gather(
                vals_v, [jnp.full((LANES,), r * KP + j, jnp.int32)])
            return tuple(
                acc[ci] + w * buf[j, pl.ds(ci * LANES, LANES)]
                for ci in range(NCH))

        acc = lax.fori_loop(
            0, KP, jbody,
            tuple(jnp.zeros((LANES,), jnp.float32) for _ in range(NCH)),
            unroll=2)
        for ci in range(NCH):
            ystage[r, pl.ds(ci * LANES, LANES)] = acc[ci]

    for k in range(LOOKAHEAD):
        issue(k * KP, k)

    def gbody(g, carry):
        for k in range(NBUF):
            r = NBUF * g + k

            @pl.when(r + LOOKAHEAD < ROWS_PW)
            def _issue_next(r=r, k=k):
                issue((r + LOOKAHEAD) * KP, (k + LOOKAHEAD) % NBUF)

            wait(k % NBUF)
            accum_row(r, bufs[k % NBUF])
        return carry

    lax.fori_loop(0, ROWS_PW // NBUF, gbody, None)

    pltpu.sync_copy(ystage, y_hbm.at[pl.ds(rbase, ROWS_PW), :])


@jax.jit
def _bag(idx_p, vals_p, table):
    mesh = plsc.VectorSubcoreMesh(core_axis_name="c", subcore_axis_name="s")

    def body(idx_hbm, vals_hbm, table_hbm, y_hbm, *scratch):
        _bag_body(idx_hbm, vals_hbm, table_hbm, y_hbm,
                  scratch[0], scratch[1], scratch[2:2 + NBUF],
                  scratch[2 + NBUF], scratch[3 + NBUF:])

    return pl.kernel(
        body,
        out_type=jax.ShapeDtypeStruct((B, CONV), jnp.float32),
        mesh=mesh,
        scratch_types=(
            [pltpu.VMEM((ROWS_PW * KP,), jnp.int32),
             pltpu.VMEM((ROWS_PW * KP,), jnp.float32)]
            + [pltpu.VMEM((KP, CONV), jnp.float32) for _ in range(NBUF)]
            + [pltpu.VMEM((ROWS_PW, CONV), jnp.float32)]
            + [pltpu.SemaphoreType.DMA for _ in range(NBUF)]
        ),
        compiler_params=pltpu.CompilerParams(needs_layout_passes=False),
    )(idx_p, vals_p, table)


def _tail_body(y_ref, wmt_ref, bm_ref, wit_ref, bi_ref, rad_ref, out_ref):
    y = y_ref[...]
    inv1 = lax.rsqrt(jnp.maximum(jnp.sum(y * y, axis=1, keepdims=True),
                                 1e-24))
    h = y * inv1
    h = jnp.dot(h, wmt_ref[...], preferred_element_type=jnp.float32,
                precision=lax.Precision.HIGHEST) + bm_ref[...]
    h = jnp.where(h >= 0, h, 0.01 * h)
    h = jnp.dot(h, wit_ref[...], preferred_element_type=jnp.float32,
                precision=lax.Precision.HIGHEST) + bi_ref[...]
    inv2 = lax.rsqrt(jnp.maximum(jnp.sum(h * h, axis=1, keepdims=True),
                                 1e-24))
    out_ref[...] = (rad_ref[0, 0] * inv2) * h


@jax.jit
def _tail(y, wmt, bm, wit, bi, rad):
    BR = 1024
    return pl.pallas_call(
        _tail_body,
        grid=(B // BR,),
        in_specs=[
            pl.BlockSpec((BR, CONV), lambda i: (i, 0)),
            pl.BlockSpec(wmt.shape, lambda i: (0, 0)),
            pl.BlockSpec(bm.shape, lambda i: (0, 0)),
            pl.BlockSpec(wit.shape, lambda i: (0, 0)),
            pl.BlockSpec(bi.shape, lambda i: (0, 0)),
            pl.BlockSpec(rad.shape, lambda i: (0, 0)),
        ],
        out_specs=pl.BlockSpec((BR, wit.shape[1]), lambda i: (i, 0)),
        out_shape=jax.ShapeDtypeStruct((B, wit.shape[1]), jnp.float32),
    )(y, wmt, bm, wit, bi, rad)


def kernel(indices, offsets, vals, table, W_mid, b_mid, W_i, b_i, radius_w):
    del offsets
    idx2 = indices.reshape(B, K).astype(jnp.int32)
    v2 = vals.reshape(B, K)
    idx_p = jnp.pad(idx2, ((0, 0), (0, KP - K))).reshape(-1)
    vals_p = jnp.pad(v2, ((0, 0), (0, KP - K))).reshape(-1)
    y = _bag(idx_p, vals_p, table)
    return _tail(y, W_mid.T, b_mid.reshape(1, -1), W_i.T,
                 b_i.reshape(1, -1), radius_w)


# speedup vs baseline: 1.0020x; 1.0020x over previous
import jax
import jax.numpy as jnp
import numpy as np
from jax import lax
from jax.experimental import pallas as pl
from jax.experimental.pallas import tpu as pltpu
from jax.experimental.pallas import tpu_sc as plsc

NC = 2
NS = 16
NW = NC * NS
LANES = 16

B = 4096
K = 50
KP = 56
CONV = 256
NCH = CONV // LANES
ROWS_PW = B // NW
NBUF = 4
GR = 8
NGRP = ROWS_PW // GR
GW = GR * KP
LOOKAHEAD = 3


def _bag_body(idx_hbm, vals_hbm, table_hbm, y_hbm,
              idx_v, vals_v, bufs, ystage, sems):
    c = lax.axis_index("c")
    s = lax.axis_index("s")
    wid = s * NC + c
    rbase = wid * ROWS_PW

    pltpu.sync_copy(idx_hbm.at[pl.ds(rbase, ROWS_PW), :], idx_v)
    pltpu.sync_copy(vals_hbm.at[pl.ds(rbase * KP, ROWS_PW * KP)], vals_v)

    def issue_lin(r, b):
        vs = [idx_v[r, pl.ds(16 * t, LANES)] for t in range(3)]
        vs.append(idx_v[r, pl.ds(KP - LANES, LANES)])
        for j in range(KP):
            t, l = divmod(j, LANES) if j < 48 else (3, j - (KP - LANES))
            iv = vs[t][l]
            pltpu.async_copy(table_hbm.at[pl.ds(iv, 1), :],
                             bufs[b].at[pl.ds(j, 1), :], sems[b])

    def issue_ind(r, b):
        pltpu.async_copy(table_hbm.at[idx_v.at[r]], bufs[b], sems[b])

    def issue(r, b):
        if b % 2 == 0:
            issue_lin(r, b)
        else:
            issue_ind(r, b)

    def wait(r, b):
        if b % 2 == 0:
            pltpu.make_async_copy(table_hbm.at[pl.ds(0, KP), :],
                                  bufs[b], sems[b]).wait()
        else:
            pltpu.make_async_copy(table_hbm.at[idx_v.at[r]],
                                  bufs[b], sems[b]).wait()

    def accum_row(r, buf):
        def jbody(j, acc):
            w = plsc.load_gather(
                vals_v, [jnp.full((LANES,), r * KP + j, jnp.int32)])
            return tuple(
                acc[ci] + w * buf[j, pl.ds(ci * LANES, LANES)]
                for ci in range(NCH))

        acc = lax.fori_loop(
            0, KP, jbody,
            tuple(jnp.zeros((LANES,), jnp.float32) for _ in range(NCH)),
            unroll=2)
        for ci in range(NCH):
            ystage[r, pl.ds(ci * LANES, LANES)] = acc[ci]

    for k in range(LOOKAHEAD):
        issue(k, k)

    def gbody(g, carry):
        for k in range(NBUF):
            r = NBUF * g + k

            @pl.when(r + LOOKAHEAD < ROWS_PW)
            def _issue_next(r=r, k=k):
                issue(r + LOOKAHEAD, (k + LOOKAHEAD) % NBUF)

            wait(r, k % NBUF)
            accum_row(r, bufs[k % NBUF])
        return carry

    lax.fori_loop(0, ROWS_PW // NBUF, gbody, None)

    pltpu.sync_copy(ystage, y_hbm.at[pl.ds(rbase, ROWS_PW), :])


@jax.jit
def _bag(idx_p, vals_p, table):
    mesh = plsc.VectorSubcoreMesh(core_axis_name="c", subcore_axis_name="s")

    def body(idx_hbm, vals_hbm, table_hbm, y_hbm, *scratch):
        _bag_body(idx_hbm, vals_hbm, table_hbm, y_hbm,
                  scratch[0], scratch[1], scratch[2:2 + NBUF],
                  scratch[2 + NBUF], scratch[3 + NBUF:])

    return pl.kernel(
        body,
        out_type=jax.ShapeDtypeStruct((B, CONV), jnp.float32),
        mesh=mesh,
        scratch_types=(
            [pltpu.VMEM((ROWS_PW, KP), jnp.int32),
             pltpu.VMEM((ROWS_PW * KP,), jnp.float32)]
            + [pltpu.VMEM((KP, CONV), jnp.float32) for _ in range(NBUF)]
            + [pltpu.VMEM((ROWS_PW, CONV), jnp.float32)]
            + [pltpu.SemaphoreType.DMA for _ in range(NBUF)]
        ),
        compiler_params=pltpu.CompilerParams(needs_layout_passes=False),
    )(idx_p, vals_p, table)


def _tail_body(y_ref, wmt_ref, bm_ref, wit_ref, bi_ref, rad_ref, out_ref):
    y = y_ref[...]
    inv1 = lax.rsqrt(jnp.maximum(jnp.sum(y * y, axis=1, keepdims=True),
                                 1e-24))
    h = y * inv1
    h = jnp.dot(h, wmt_ref[...], preferred_element_type=jnp.float32,
                precision=lax.Precision.HIGHEST) + bm_ref[...]
    h = jnp.where(h >= 0, h, 0.01 * h)
    h = jnp.dot(h, wit_ref[...], preferred_element_type=jnp.float32,
                precision=lax.Precision.HIGHEST) + bi_ref[...]
    inv2 = lax.rsqrt(jnp.maximum(jnp.sum(h * h, axis=1, keepdims=True),
                                 1e-24))
    out_ref[...] = (rad_ref[0, 0] * inv2) * h


@jax.jit
def _tail(y, wmt, bm, wit, bi, rad):
    BR = 1024
    return pl.pallas_call(
        _tail_body,
        grid=(B // BR,),
        in_specs=[
            pl.BlockSpec((BR, CONV), lambda i: (i, 0)),
            pl.BlockSpec(wmt.shape, lambda i: (0, 0)),
            pl.BlockSpec(bm.shape, lambda i: (0, 0)),
            pl.BlockSpec(wit.shape, lambda i: (0, 0)),
            pl.BlockSpec(bi.shape, lambda i: (0, 0)),
            pl.BlockSpec(rad.shape, lambda i: (0, 0)),
        ],
        out_specs=pl.BlockSpec((BR, wit.shape[1]), lambda i: (i, 0)),
        out_shape=jax.ShapeDtypeStruct((B, wit.shape[1]), jnp.float32),
    )(y, wmt, bm, wit, bi, rad)


def kernel(indices, offsets, vals, table, W_mid, b_mid, W_i, b_i, radius_w):
    del offsets
    idx2 = indices.reshape(B, K).astype(jnp.int32)
    v2 = vals.reshape(B, K)
    idx_p = jnp.pad(idx2, ((0, 0), (0, KP - K)))
    vals_p = jnp.pad(v2, ((0, 0), (0, KP - K))).reshape(-1)
    y = _bag(idx_p, vals_p, table)
    return _tail(y, W_mid.T, b_mid.reshape(1, -1), W_i.T,
                 b_i.reshape(1, -1), radius_w)


# speedup vs baseline: 5.6882x; 5.6768x over previous
import jax
import jax.numpy as jnp
import numpy as np
from jax import lax
from jax.experimental import pallas as pl
from jax.experimental.pallas import tpu as pltpu
from jax.experimental.pallas import tpu_sc as plsc

NC = 2
NS = 16
NW = NC * NS
LANES = 16

B = 4096
K = 50
KP = K
CONV = 256
NCH = CONV // LANES
ROWS_PW = B // NW
NBUF = 4
GR = 8
NGRP = ROWS_PW // GR
GW = GR * KP
LOOKAHEAD = 3


def _bag_body(idx_hbm, vals_hbm, table_hbm, y_hbm,
              idx_v, vals_v, bufs, ystage, wdummy, sems):
    c = lax.axis_index("c")
    s = lax.axis_index("s")
    wid = s * NC + c
    rbase = wid * ROWS_PW

    pltpu.sync_copy(idx_hbm.at[pl.ds(rbase * KP, ROWS_PW * KP)], idx_v)
    pltpu.sync_copy(vals_hbm.at[pl.ds(rbase * KP, ROWS_PW * KP)], vals_v)

    def issue(r, b):
        base = r * KP
        vs = [idx_v[pl.ds(base + 16 * t, LANES)] for t in range(3)]
        vs.append(idx_v[pl.ds(base + KP - LANES, LANES)])
        for j in range(KP):
            t, l = divmod(j, LANES) if j < 48 else (3, j - (KP - LANES))
            iv = vs[t][l]
            pltpu.async_copy(table_hbm.at[pl.ds(iv, 1), :],
                             bufs[b].at[pl.ds(j, 1), :], sems[b])

    def wait(r, b):
        pltpu.make_async_copy(vals_hbm.at[pl.ds(0, KP * CONV)],
                              wdummy, sems[b]).wait()

    def accum_row(r, buf):
        def jbody(j, acc):
            w = plsc.load_gather(
                vals_v, [jnp.full((LANES,), r * KP + j, jnp.int32)])
            return tuple(
                acc[ci] + w * buf[j, pl.ds(ci * LANES, LANES)]
                for ci in range(NCH))

        acc = lax.fori_loop(
            0, KP, jbody,
            tuple(jnp.zeros((LANES,), jnp.float32) for _ in range(NCH)),
            unroll=2)
        for ci in range(NCH):
            ystage[r, pl.ds(ci * LANES, LANES)] = acc[ci]

    for k in range(LOOKAHEAD):
        issue(k, k)

    def gbody(g, carry):
        for k in range(NBUF):
            r = NBUF * g + k

            @pl.when(r + LOOKAHEAD < ROWS_PW)
            def _issue_next(r=r, k=k):
                issue(r + LOOKAHEAD, (k + LOOKAHEAD) % NBUF)

            wait(r, k % NBUF)
            accum_row(r, bufs[k % NBUF])
        return carry

    lax.fori_loop(0, ROWS_PW // NBUF, gbody, None)

    pltpu.sync_copy(ystage, y_hbm.at[pl.ds(rbase, ROWS_PW), :])


@jax.jit
def _bag(idx_p, vals_p, table):
    mesh = plsc.VectorSubcoreMesh(core_axis_name="c", subcore_axis_name="s")

    def body(idx_hbm, vals_hbm, table_hbm, y_hbm, *scratch):
        _bag_body(idx_hbm, vals_hbm, table_hbm, y_hbm,
                  scratch[0], scratch[1], scratch[2:2 + NBUF],
                  scratch[2 + NBUF], scratch[3 + NBUF],
                  scratch[4 + NBUF:])

    return pl.kernel(
        body,
        out_type=jax.ShapeDtypeStruct((B, CONV), jnp.float32),
        mesh=mesh,
        scratch_types=(
            [pltpu.VMEM((ROWS_PW * KP,), jnp.int32),
             pltpu.VMEM((ROWS_PW * KP,), jnp.float32)]
            + [pltpu.VMEM((KP, CONV), jnp.float32) for _ in range(NBUF)]
            + [pltpu.VMEM((ROWS_PW, CONV), jnp.float32)]
            + [pltpu.VMEM((KP * CONV,), jnp.float32)]
            + [pltpu.SemaphoreType.DMA for _ in range(NBUF)]
        ),
        compiler_params=pltpu.CompilerParams(needs_layout_passes=False),
    )(idx_p, vals_p, table)


def _tail_body(y_ref, wmt_ref, bm_ref, wit_ref, bi_ref, rad_ref, out_ref):
    y = y_ref[...]
    inv1 = lax.rsqrt(jnp.maximum(jnp.sum(y * y, axis=1, keepdims=True),
                                 1e-24))
    h = y * inv1
    h = jnp.dot(h, wmt_ref[...], preferred_element_type=jnp.float32,
                precision=lax.Precision.HIGHEST) + bm_ref[...]
    h = jnp.where(h >= 0, h, 0.01 * h)
    h = jnp.dot(h, wit_ref[...], preferred_element_type=jnp.float32,
                precision=lax.Precision.HIGHEST) + bi_ref[...]
    inv2 = lax.rsqrt(jnp.maximum(jnp.sum(h * h, axis=1, keepdims=True),
                                 1e-24))
    out_ref[...] = (rad_ref[0, 0] * inv2) * h


@jax.jit
def _tail(y, wmt, bm, wit, bi, rad):
    BR = 1024
    return pl.pallas_call(
        _tail_body,
        grid=(B // BR,),
        in_specs=[
            pl.BlockSpec((BR, CONV), lambda i: (i, 0)),
            pl.BlockSpec(wmt.shape, lambda i: (0, 0)),
            pl.BlockSpec(bm.shape, lambda i: (0, 0)),
            pl.BlockSpec(wit.shape, lambda i: (0, 0)),
            pl.BlockSpec(bi.shape, lambda i: (0, 0)),
            pl.BlockSpec(rad.shape, lambda i: (0, 0)),
        ],
        out_specs=pl.BlockSpec((BR, wit.shape[1]), lambda i: (i, 0)),
        out_shape=jax.ShapeDtypeStruct((B, wit.shape[1]), jnp.float32),
    )(y, wmt, bm, wit, bi, rad)


def kernel(indices, offsets, vals, table, W_mid, b_mid, W_i, b_i, radius_w):
    del offsets
    y = _bag(indices.astype(jnp.int32), vals, table)
    return _tail(y, W_mid.T, b_mid.reshape(1, -1), W_i.T,
                 b_i.reshape(1, -1), radius_w)


# speedup vs baseline: 5.7769x; 1.0156x over previous
import jax
import jax.numpy as jnp
import numpy as np
from jax import lax
from jax.experimental import pallas as pl
from jax.experimental.pallas import tpu as pltpu
from jax.experimental.pallas import tpu_sc as plsc

NC = 2
NS = 16
NW = NC * NS
LANES = 16

B = 4096
K = 50
KP = K
CONV = 256
NCH = CONV // LANES
ROWS_PW = B // NW
NBUF = 5
LOOKAHEAD = NBUF - 1


def _bag_body(idx_hbm, vals_hbm, table_hbm, y_hbm,
              idx_v, vals_v, bufs, ystage, wdummy, sems):
    c = lax.axis_index("c")
    s = lax.axis_index("s")
    wid = s * NC + c
    rbase = wid * ROWS_PW

    pltpu.sync_copy(idx_hbm.at[pl.ds(rbase * KP, ROWS_PW * KP)], idx_v)
    pltpu.sync_copy(vals_hbm.at[pl.ds(rbase * KP, ROWS_PW * KP)], vals_v)

    def issue(r, b):
        base = r * KP
        vs = [idx_v[pl.ds(base + 16 * t, LANES)] for t in range(3)]
        vs.append(idx_v[pl.ds(base + KP - LANES, LANES)])
        for j in range(KP):
            t, l = divmod(j, LANES) if j < 48 else (3, j - (KP - LANES))
            iv = vs[t][l]
            pltpu.async_copy(table_hbm.at[pl.ds(iv, 1), :],
                             bufs[b].at[pl.ds(j, 1), :], sems[b])

    def wait(r, b):
        pltpu.make_async_copy(vals_hbm.at[pl.ds(0, KP * CONV)],
                              wdummy, sems[b]).wait()

    def accum_row(r, buf):
        def jbody(j, acc):
            w = plsc.load_gather(
                vals_v, [jnp.full((LANES,), r * KP + j, jnp.int32)])
            return tuple(
                acc[ci] + w * buf[j, pl.ds(ci * LANES, LANES)]
                for ci in range(NCH))

        acc = lax.fori_loop(
            0, KP, jbody,
            tuple(jnp.zeros((LANES,), jnp.float32) for _ in range(NCH)),
            unroll=2)
        for ci in range(NCH):
            ystage[r, pl.ds(ci * LANES, LANES)] = acc[ci]

    for k in range(LOOKAHEAD):
        issue(k, k)

    def gbody(g, carry):
        for k in range(NBUF):
            r = NBUF * g + k

            @pl.when(r + LOOKAHEAD < ROWS_PW)
            def _issue_next(r=r, k=k):
                issue(r + LOOKAHEAD, (k + LOOKAHEAD) % NBUF)

            wait(r, k % NBUF)
            accum_row(r, bufs[k % NBUF])
        return carry

    nfull = ROWS_PW // NBUF
    lax.fori_loop(0, nfull, gbody, None)
    for k in range(ROWS_PW - nfull * NBUF):
        r = nfull * NBUF + k
        wait(r, k % NBUF)
        accum_row(r, bufs[k % NBUF])

    pltpu.sync_copy(ystage, y_hbm.at[pl.ds(rbase, ROWS_PW), :])


@jax.jit
def _bag(idx_p, vals_p, table):
    mesh = plsc.VectorSubcoreMesh(core_axis_name="c", subcore_axis_name="s")

    def body(idx_hbm, vals_hbm, table_hbm, y_hbm, *scratch):
        _bag_body(idx_hbm, vals_hbm, table_hbm, y_hbm,
                  scratch[0], scratch[1], scratch[2:2 + NBUF],
                  scratch[2 + NBUF], scratch[3 + NBUF],
                  scratch[4 + NBUF:])

    return pl.kernel(
        body,
        out_type=jax.ShapeDtypeStruct((B, CONV), jnp.float32),
        mesh=mesh,
        scratch_types=(
            [pltpu.VMEM((ROWS_PW * KP,), jnp.int32),
             pltpu.VMEM((ROWS_PW * KP,), jnp.float32)]
            + [pltpu.VMEM((KP, CONV), jnp.float32) for _ in range(NBUF)]
            + [pltpu.VMEM((ROWS_PW, CONV), jnp.float32)]
            + [pltpu.VMEM((KP * CONV,), jnp.float32)]
            + [pltpu.SemaphoreType.DMA for _ in range(NBUF)]
        ),
        compiler_params=pltpu.CompilerParams(needs_layout_passes=False),
    )(idx_p, vals_p, table)


def _tail_body(y_ref, wmt_ref, bm_ref, wit_ref, bi_ref, rad_ref, out_ref):
    y = y_ref[...]
    inv1 = lax.rsqrt(jnp.maximum(jnp.sum(y * y, axis=1, keepdims=True),
                                 1e-24))
    h = y * inv1
    h = jnp.dot(h, wmt_ref[...], preferred_element_type=jnp.float32,
                precision=lax.Precision.HIGHEST) + bm_ref[...]
    h = jnp.where(h >= 0, h, 0.01 * h)
    h = jnp.dot(h, wit_ref[...], preferred_element_type=jnp.float32,
                precision=lax.Precision.HIGHEST) + bi_ref[...]
    inv2 = lax.rsqrt(jnp.maximum(jnp.sum(h * h, axis=1, keepdims=True),
                                 1e-24))
    out_ref[...] = (rad_ref[0, 0] * inv2) * h


@jax.jit
def _tail(y, wmt, bm, wit, bi, rad):
    BR = 1024
    return pl.pallas_call(
        _tail_body,
        grid=(B // BR,),
        in_specs=[
            pl.BlockSpec((BR, CONV), lambda i: (i, 0)),
            pl.BlockSpec(wmt.shape, lambda i: (0, 0)),
            pl.BlockSpec(bm.shape, lambda i: (0, 0)),
            pl.BlockSpec(wit.shape, lambda i: (0, 0)),
            pl.BlockSpec(bi.shape, lambda i: (0, 0)),
            pl.BlockSpec(rad.shape, lambda i: (0, 0)),
        ],
        out_specs=pl.BlockSpec((BR, wit.shape[1]), lambda i: (i, 0)),
        out_shape=jax.ShapeDtypeStruct((B, wit.shape[1]), jnp.float32),
    )(y, wmt, bm, wit, bi, rad)


def kernel(indices, offsets, vals, table, W_mid, b_mid, W_i, b_i, radius_w):
    del offsets
    y = _bag(indices.astype(jnp.int32), vals, table)
    return _tail(y, W_mid.T, b_mid.reshape(1, -1), W_i.T,
                 b_i.reshape(1, -1), radius_w)
